# baseline (speedup 1.0000x reference)
import jax
import jax.numpy as jnp
import numpy as np
from jax.experimental import pallas as pl
from jax.experimental.pallas import tpu as pltpu

_N, _C, _H, _W = 32, 256, 64, 64
_HW = _H * _W
_EPS = 1e-05

_PERM_NP = np.asarray(
    [31, 7, 4, 29, 16, 19, 2, 5, 30, 3, 22, 6, 18, 10, 11, 15,
     20, 8, 24, 9, 25, 13, 14, 17, 23, 0, 21, 26, 1, 28, 27, 12],
    dtype=np.int32)


def _cycle_plan(perm):
    n = len(perm)
    visited = [False] * n
    load, out_idx, emit_normal, emit_first, save_first = [], [], [], [], []
    prev_cycle_last = None
    for s in range(n):
        if visited[s]:
            continue
        cyc = []
        a = s
        while not visited[a]:
            visited[a] = True
            cyc.append(a)
            a = int(perm[a])
        for m, a in enumerate(cyc):
            load.append(a)
            if m == 0:
                save_first.append(1)
                emit_normal.append(0)
                if prev_cycle_last is None:
                    emit_first.append(0)
                    out_idx.append(-1)
                else:
                    emit_first.append(1)
                    out_idx.append(prev_cycle_last)
            else:
                save_first.append(0)
                emit_first.append(0)
                emit_normal.append(1)
                out_idx.append(cyc[m - 1])
        prev_cycle_last = cyc[-1]
    load.append(load[-1])
    save_first.append(0)
    emit_normal.append(0)
    emit_first.append(1)
    out_idx.append(prev_cycle_last)
    out_idx[0] = out_idx[1]
    idx = np.asarray([load, out_idx], dtype=np.int32)
    flg = np.asarray([emit_normal, emit_first, save_first], dtype=np.int32)
    return idx, flg


_IDX_NP, _FLG_NP = _cycle_plan(_PERM_NP)
_STEPS = _IDX_NP.shape[1]


_HH = _HW // 2


def _body(idx_ref, flg_ref, x0_ref, x1_ref, o_ref,
          xfirst, first_m, first_r, prev_m, prev_s):
    t = pl.program_id(0)
    for half, xh in ((0, x0_ref), (1, x1_ref)):
        sl = pl.ds(half * _HH, _HH)
        cur = xh[...]
        s = jnp.sum(cur, axis=0).reshape(1, _HH)
        sq = jnp.sum(cur * cur, axis=0).reshape(1, _HH)
        m_cur = s * (1.0 / _C)
        var = (sq - _C * m_cur * m_cur) * (1.0 / (_C - 1))
        s_cur = jnp.sqrt(var + _EPS)
        r_cur = 1.0 / s_cur

        @pl.when(flg_ref[0, t] == 1)
        def _():
            f = r_cur * prev_s[:, sl]
            g = prev_m[:, sl] - m_cur * f
            o_ref[:, sl] = cur * f + g

        @pl.when(flg_ref[1, t] == 1)
        def _():
            f = first_r[:, sl] * prev_s[:, sl]
            g = prev_m[:, sl] - first_m[:, sl] * f
            o_ref[:, sl] = xfirst[:, sl] * f + g

        @pl.when(flg_ref[2, t] == 1)
        def _():
            xfirst[:, sl] = cur
            first_m[:, sl] = m_cur
            first_r[:, sl] = r_cur

        prev_m[:, sl] = m_cur
        prev_s[:, sl] = s_cur


def kernel(x):
    n, c, h, w = x.shape
    xr = x.reshape(n, c, h * w)
    idx = jnp.asarray(_IDX_NP)
    flg = jnp.asarray(_FLG_NP)
    out = pl.pallas_call(
        _body,
        grid_spec=pltpu.PrefetchScalarGridSpec(
            num_scalar_prefetch=2,
            grid=(_STEPS,),
            in_specs=[
                pl.BlockSpec((None, c, _HH), lambda t, i, f: (i[0, t], 0, 0)),
                pl.BlockSpec((None, c, _HH), lambda t, i, f: (i[0, t], 0, 1)),
            ],
            out_specs=pl.BlockSpec((None, c, _HW), lambda t, i, f: (i[1, t], 0, 0)),
            scratch_shapes=[
                pltpu.VMEM((c, _HW), jnp.float32),
                pltpu.VMEM((1, _HW), jnp.float32),
                pltpu.VMEM((1, _HW), jnp.float32),
                pltpu.VMEM((1, _HW), jnp.float32),
                pltpu.VMEM((1, _HW), jnp.float32),
            ],
        ),
        out_shape=jax.ShapeDtypeStruct((n, c, h * w), jnp.float32),
    )(idx, flg, xr, xr)

    return out.reshape(n, c, h, w)


# speedup vs baseline: 1.0181x; 1.0181x over previous
import jax
import jax.numpy as jnp
import numpy as np
from jax.experimental import pallas as pl
from jax.experimental.pallas import tpu as pltpu

_N, _C, _H, _W = 32, 256, 64, 64
_HW = _H * _W
_EPS = 1e-05
_CK = 1024
_NK = _HW // _CK
_BI = 6
_BO = 4

_PERM = [31, 7, 4, 29, 16, 19, 2, 5, 30, 3, 22, 6, 18, 10, 11, 15,
         20, 8, 24, 9, 25, 13, 14, 17, 23, 0, 21, 26, 1, 28, 27, 12]


def _cycles(perm):
    n, visited, cycles = len(perm), [False] * len(perm), []
    for s in range(n):
        if visited[s]:
            continue
        cyc, a = [], s
        while not visited[a]:
            visited[a] = True
            cyc.append(a)
            a = perm[a]
        cycles.append(cyc)
    return cycles


_CYCLES = _cycles(_PERM)


def _stats(cur):
    s = jnp.sum(cur, axis=0).reshape(1, _CK)
    sq = jnp.sum(cur * cur, axis=0).reshape(1, _CK)
    m = s * (1.0 / _C)
    var = (sq - _C * m * m) * (1.0 / (_C - 1))
    sd = jnp.sqrt(var + _EPS)
    return m, sd, 1.0 / sd


def _body(x_hbm, o_hbm, inbuf, outbuf, xfirst, sem_i, sem_o):
    loads = []
    for cyc in _CYCLES:
        for m, a in enumerate(cyc):
            for k in range(_NK):
                loads.append((a, k, m, cyc))

    def in_cp(item, slot):
        a, k, _, _ = loads[item]
        return pltpu.make_async_copy(
            x_hbm.at[a, :, pl.ds(k * _CK, _CK)], inbuf.at[slot],
            sem_i.at[slot])

    def out_cp(dst, k, slot):
        return pltpu.make_async_copy(
            outbuf.at[slot], o_hbm.at[dst, :, pl.ds(k * _CK, _CK)],
            sem_o.at[slot])

    for i in range(_BI):
        in_cp(i, i).start()

    prev_m = [None] * _NK
    prev_s = [None] * _NK
    first_m = [None] * _NK
    first_r = [None] * _NK
    n_out = 0

    def emit(dst, k, f, g, src):
        nonlocal n_out
        slot = n_out % _BO
        if n_out >= _BO:
            out_cp(0, 0, slot).wait()
        outbuf[slot] = src * f + g
        out_cp(dst, k, slot).start()
        n_out += 1

    for i, (a, k, m, cyc) in enumerate(loads):
        slot = i % _BI
        in_cp(i, slot).wait()
        cur = inbuf[slot]
        m_cur, s_cur, r_cur = _stats(cur)

        if m == 0:
            xfirst[:, pl.ds(k * _CK, _CK)] = cur
            first_m[k], first_r[k] = m_cur, r_cur
        else:
            f = r_cur * prev_s[k]
            g = prev_m[k] - m_cur * f
            emit(cyc[m - 1], k, f, g, cur)

        if m == len(cyc) - 1:
            fm, fr = first_m[k], first_r[k]
            f = fr * s_cur
            g = m_cur - fm * f
            emit(cyc[-1], k, f, g, xfirst[:, pl.ds(k * _CK, _CK)])

        prev_m[k], prev_s[k] = m_cur, s_cur

        nxt = i + _BI
        if nxt < len(loads):
            in_cp(nxt, slot).start()

    for j in range(min(n_out, _BO)):
        out_cp(0, 0, j).wait()


def kernel(x):
    n, c, h, w = x.shape
    xr = x.reshape(n, c, h * w)
    out = pl.pallas_call(
        _body,
        grid=(1,),
        in_specs=[pl.BlockSpec(memory_space=pl.ANY)],
        out_specs=pl.BlockSpec(memory_space=pl.ANY),
        scratch_shapes=[
            pltpu.VMEM((_BI, c, _CK), jnp.float32),
            pltpu.VMEM((_BO, c, _CK), jnp.float32),
            pltpu.VMEM((c, _HW), jnp.float32),
            pltpu.SemaphoreType.DMA((_BI,)),
            pltpu.SemaphoreType.DMA((_BO,)),
        ],
        out_shape=jax.ShapeDtypeStruct((n, c, h * w), jnp.float32),
    )(xr)
    return out.reshape(n, c, h, w)
